# Initial kernel scaffold; baseline (speedup 1.0000x reference)
import functools

import jax
import jax.numpy as jnp
import numpy as np
from jax.experimental import pallas as pl
from jax.experimental.pallas import tpu as pltpu

TASKS = 3
EPS = float(np.finfo(np.float64).eps)


def _moe_kernel(x_ref, wg_ref, w1_ref, b1_ref, w2_ref, b2_ref,
                out_ref, gates_ref, *, n_experts):
    e = pl.program_id(1)

    @pl.when(e == 0)
    def _compute_gates():
        x = x_ref[...]
        for t in range(TASKS):
            logits = jax.lax.dot_general(
                x, wg_ref[t],
                (((1,), (0,)), ((), ())),
                preferred_element_type=jnp.float32)
            m1 = jnp.max(logits, axis=-1, keepdims=True)
            eq1 = logits == m1
            first1 = jnp.logical_and(eq1, jnp.cumsum(eq1.astype(jnp.int32), axis=-1) == 1)
            l2 = jnp.where(first1, -jnp.inf, logits)
            m2 = jnp.max(l2, axis=-1, keepdims=True)
            eq2 = l2 == m2
            first2 = jnp.logical_and(eq2, jnp.cumsum(eq2.astype(jnp.int32), axis=-1) == 1)
            z = jnp.exp(m2 - m1)
            g1 = 1.0 / (1.0 + z)
            g2 = z / (1.0 + z)
            gates = jnp.where(first1, g1, 0.0) + jnp.where(first2, g2, 0.0)
            gates = jnp.where(gates <= 0.0001, 0.0, gates)
            gates_ref[t] = gates

    x = x_ref[...]
    w1 = w1_ref[0]
    w2 = w2_ref[0]
    h = jax.lax.dot_general(x, w1, (((1,), (1,)), ((), ())),
                            preferred_element_type=jnp.float32)
    h = jax.nn.relu(h + b1_ref[0][None, :])
    y = jax.lax.dot_general(h, w2, (((1,), (1,)), ((), ())),
                            preferred_element_type=jnp.float32)
    y = y + b2_ref[0][None, :]
    ey = jnp.exp(y)

    contrib = jnp.stack(
        [gates_ref[t, :, e][:, None] * ey for t in range(TASKS)], axis=0)

    @pl.when(e == 0)
    def _init():
        out_ref[...] = contrib

    @pl.when(e > 0)
    def _acc():
        out_ref[...] += contrib

    @pl.when(e == n_experts - 1)
    def _finish():
        acc = out_ref[...]
        out_ref[...] = jnp.log(jnp.where(acc == 0.0, EPS, acc))


def kernel(x, w_gate, fc1_w, fc1_b, fc2_w, fc2_b):
    B, D = x.shape
    E, H, _ = fc1_w.shape
    O = fc2_w.shape[1]
    BT = 256
    n_b = B // BT

    grid = (n_b, E)
    out = pl.pallas_call(
        functools.partial(_moe_kernel, n_experts=E),
        grid=grid,
        in_specs=[
            pl.BlockSpec((BT, D), lambda i, e: (i, 0)),
            pl.BlockSpec((TASKS, D, E), lambda i, e: (0, 0, 0)),
            pl.BlockSpec((1, H, D), lambda i, e: (e, 0, 0)),
            pl.BlockSpec((1, H), lambda i, e: (e, 0)),
            pl.BlockSpec((1, O, H), lambda i, e: (e, 0, 0)),
            pl.BlockSpec((1, O), lambda i, e: (e, 0)),
        ],
        out_specs=pl.BlockSpec((TASKS, BT, O), lambda i, e: (0, i, 0)),
        out_shape=jax.ShapeDtypeStruct((TASKS, B, O), jnp.float32),
        scratch_shapes=[pltpu.VMEM((TASKS, BT, E), jnp.float32)],
    )(x, w_gate, fc1_w, fc1_b, fc2_w, fc2_b)
    return out


# speedup vs baseline: 1.2705x; 1.2705x over previous
import functools

import jax
import jax.numpy as jnp
import numpy as np
from jax.experimental import pallas as pl
from jax.experimental.pallas import tpu as pltpu

TASKS = 3
EPS = float(np.finfo(np.float64).eps)


def _moe_kernel(x_ref, wg_ref, w1_ref, b1_ref, w2_ref, b2_ref,
                out_ref, gates_ref, *, n_experts):
    e = pl.program_id(1)

    @pl.when(e == 0)
    def _compute_gates():
        x = x_ref[...]
        for t in range(TASKS):
            logits = jax.lax.dot_general(
                x, wg_ref[t],
                (((1,), (0,)), ((), ())),
                preferred_element_type=jnp.float32)
            idx = jax.lax.broadcasted_iota(jnp.int32, logits.shape, 1)
            m1 = jnp.max(logits, axis=-1, keepdims=True)
            eq1 = logits == m1
            i1 = jnp.min(jnp.where(eq1, idx, 127), axis=-1, keepdims=True)
            first1 = idx == i1
            l2 = jnp.where(first1, -jnp.inf, logits)
            m2 = jnp.max(l2, axis=-1, keepdims=True)
            eq2 = l2 == m2
            i2 = jnp.min(jnp.where(eq2, idx, 127), axis=-1, keepdims=True)
            first2 = idx == i2
            z = jnp.exp(m2 - m1)
            g1 = 1.0 / (1.0 + z)
            g2 = z / (1.0 + z)
            gates = jnp.where(first1, g1, 0.0) + jnp.where(first2, g2, 0.0)
            gates = jnp.where(gates <= 0.0001, 0.0, gates)
            gates_ref[t] = gates

    x = x_ref[...]
    w1 = w1_ref[0]
    w2 = w2_ref[0]
    h = jax.lax.dot_general(x, w1, (((1,), (1,)), ((), ())),
                            preferred_element_type=jnp.float32)
    h = jax.nn.relu(h + b1_ref[0])
    y = jax.lax.dot_general(h, w2, (((1,), (1,)), ((), ())),
                            preferred_element_type=jnp.float32)
    y = y + b2_ref[0]
    ey = jnp.exp(y)

    gates_all = gates_ref[...]
    eidx = jax.lax.broadcasted_iota(jnp.int32, gates_all.shape, 2)
    ge = jnp.sum(jnp.where(eidx == e, gates_all, 0.0), axis=-1)
    contrib = ge[:, :, None] * ey[None, :, :]

    @pl.when(e == 0)
    def _init():
        out_ref[...] = contrib

    @pl.when(e > 0)
    def _acc():
        out_ref[...] += contrib

    @pl.when(e == n_experts - 1)
    def _finish():
        acc = out_ref[...]
        out_ref[...] = jnp.log(jnp.where(acc == 0.0, EPS, acc))


def kernel(x, w_gate, fc1_w, fc1_b, fc2_w, fc2_b):
    B, D = x.shape
    E, H, _ = fc1_w.shape
    O = fc2_w.shape[1]
    BT = 256
    n_b = B // BT

    grid = (n_b, E)
    out = pl.pallas_call(
        functools.partial(_moe_kernel, n_experts=E),
        grid=grid,
        in_specs=[
            pl.BlockSpec((BT, D), lambda i, e: (i, 0)),
            pl.BlockSpec((TASKS, D, E), lambda i, e: (0, 0, 0)),
            pl.BlockSpec((1, H, D), lambda i, e: (e, 0, 0)),
            pl.BlockSpec((1, 1, H), lambda i, e: (e, 0, 0)),
            pl.BlockSpec((1, O, H), lambda i, e: (e, 0, 0)),
            pl.BlockSpec((1, 1, O), lambda i, e: (e, 0, 0)),
        ],
        out_specs=pl.BlockSpec((TASKS, BT, O), lambda i, e: (0, i, 0)),
        out_shape=jax.ShapeDtypeStruct((TASKS, B, O), jnp.float32),
        scratch_shapes=[pltpu.VMEM((TASKS, BT, E), jnp.float32)],
    )(x, w_gate, fc1_w, fc1_b.reshape(E, 1, H), fc2_w, fc2_b.reshape(E, 1, O))
    return out


# speedup vs baseline: 2.0362x; 1.6026x over previous
import functools

import jax
import jax.numpy as jnp
import numpy as np
from jax.experimental import pallas as pl
from jax.experimental.pallas import tpu as pltpu

TASKS = 3
EPS = float(np.finfo(np.float64).eps)


def _moe_kernel(x_ref, wg_ref, w1_ref, b1_ref, w2_ref, b2_ref,
                out_ref, gates_ref, *, n_experts):
    e = pl.program_id(1)

    @pl.when(e == 0)
    def _compute_gates():
        x = x_ref[...]
        for t in range(TASKS):
            logits = jax.lax.dot_general(
                x, wg_ref[t],
                (((1,), (0,)), ((), ())),
                preferred_element_type=jnp.float32)
            idx = jax.lax.broadcasted_iota(jnp.int32, logits.shape, 1)
            m1 = jnp.max(logits, axis=-1, keepdims=True)
            eq1 = logits == m1
            i1 = jnp.min(jnp.where(eq1, idx, 127), axis=-1, keepdims=True)
            first1 = idx == i1
            l2 = jnp.where(first1, -jnp.inf, logits)
            m2 = jnp.max(l2, axis=-1, keepdims=True)
            eq2 = l2 == m2
            i2 = jnp.min(jnp.where(eq2, idx, 127), axis=-1, keepdims=True)
            first2 = idx == i2
            z = jnp.exp(m2 - m1)
            g1 = 1.0 / (1.0 + z)
            g2 = z / (1.0 + z)
            gates = jnp.where(first1, g1, 0.0) + jnp.where(first2, g2, 0.0)
            gates = jnp.where(gates <= 0.0001, 0.0, gates)
            gates_ref[t] = gates

    x = x_ref[...]
    w1 = w1_ref[0]
    w2 = w2_ref[0]
    h = jax.lax.dot_general(x, w1, (((1,), (1,)), ((), ())),
                            preferred_element_type=jnp.float32)
    h = jax.nn.relu(h + b1_ref[0])
    y = jax.lax.dot_general(h, w2, (((1,), (1,)), ((), ())),
                            preferred_element_type=jnp.float32)
    y = y + b2_ref[0]
    ey = jnp.exp(y)

    gates_all = gates_ref[...]
    eidx = jax.lax.broadcasted_iota(jnp.int32, gates_all.shape, 2)
    ge = jnp.sum(jnp.where(eidx == e, gates_all, 0.0), axis=-1)
    contrib = ge[:, :, None] * ey[None, :, :]

    @pl.when(e == 0)
    def _init():
        out_ref[...] = contrib

    @pl.when(e > 0)
    def _acc():
        out_ref[...] += contrib

    @pl.when(e == n_experts - 1)
    def _finish():
        acc = out_ref[...]
        out_ref[...] = jnp.log(jnp.where(acc == 0.0, EPS, acc))


def kernel(x, w_gate, fc1_w, fc1_b, fc2_w, fc2_b):
    B, D = x.shape
    E, H, _ = fc1_w.shape
    O = fc2_w.shape[1]
    BT = 512
    n_b = B // BT

    grid = (n_b, E)
    out = pl.pallas_call(
        functools.partial(_moe_kernel, n_experts=E),
        grid=grid,
        in_specs=[
            pl.BlockSpec((BT, D), lambda i, e: (i, 0)),
            pl.BlockSpec((TASKS, D, E), lambda i, e: (0, 0, 0)),
            pl.BlockSpec((1, H, D), lambda i, e: (e, 0, 0)),
            pl.BlockSpec((1, 1, H), lambda i, e: (e, 0, 0)),
            pl.BlockSpec((1, O, H), lambda i, e: (e, 0, 0)),
            pl.BlockSpec((1, 1, O), lambda i, e: (e, 0, 0)),
        ],
        out_specs=pl.BlockSpec((TASKS, BT, O), lambda i, e: (0, i, 0)),
        out_shape=jax.ShapeDtypeStruct((TASKS, B, O), jnp.float32),
        scratch_shapes=[pltpu.VMEM((TASKS, BT, E), jnp.float32)],
        compiler_params=pltpu.CompilerParams(
            vmem_limit_bytes=63 * 1024 * 1024),
    )(x, w_gate, fc1_w, fc1_b.reshape(E, 1, H), fc2_w, fc2_b.reshape(E, 1, O))
    return out
